# Initial kernel scaffold; baseline (speedup 1.0000x reference)
import jax
import jax.numpy as jnp
from jax.experimental import pallas as pl


def kernel(nodes, senders, receivers, edges):
    raise NotImplementedError("write your pallas kernel here")


# speedup vs baseline: 1.9609x; 1.9609x over previous
import functools

import jax
import jax.numpy as jnp
from jax import lax
from jax.experimental import pallas as pl
from jax.experimental.pallas import tpu as pltpu
from jax.experimental.pallas import tpu_sc as plsc

_SLOTS = 4
_NWORKERS = 32
_CHUNK = 80


def _ranks(keys, E):
    order = jnp.argsort(keys, stable=True)
    sk = keys[order]
    iota = jnp.arange(E, dtype=jnp.int32)
    newseg = jnp.concatenate([jnp.ones((1,), jnp.bool_), sk[1:] != sk[:-1]])
    first = lax.cummax(jnp.where(newseg, iota, 0))
    rank_sorted = iota - first
    return jnp.zeros((E,), jnp.int32).at[order].set(rank_sorted)


def _make_gather_kernel(nrows, df, de, n_slots_flat):
    nchunks = n_slots_flat // _CHUNK
    iters = (nchunks + _NWORKERS - 1) // _NWORKERS
    mesh = plsc.VectorSubcoreMesh(core_axis_name="c", subcore_axis_name="s")

    @functools.partial(
        pl.kernel,
        out_type=(
            jax.ShapeDtypeStruct((n_slots_flat, df), jnp.float32),
            jax.ShapeDtypeStruct((n_slots_flat, de), jnp.float32),
        ),
        mesh=mesh,
        compiler_params=pltpu.CompilerParams(use_tc_tiling_on_sc=False),
        scratch_types=[
            pltpu.VMEM((_CHUNK,), jnp.int32),
            pltpu.VMEM((_CHUNK,), jnp.int32),
            pltpu.VMEM((_CHUNK, df), jnp.float32),
            pltpu.VMEM((_CHUNK, de), jnp.float32),
            pltpu.SemaphoreType.DMA,
            pltpu.SemaphoreType.DMA,
        ],
    )
    def k(nodes_hbm, edges_hbm, idxv_hbm, idxe_hbm, neigh_hbm, edg_hbm,
          idxv_v, idxe_v, nbuf, ebuf, semn, seme):
        wid = lax.axis_index("s") * 2 + lax.axis_index("c")
        for it in range(iters):
            c = wid + _NWORKERS * it

            @pl.when(c < nchunks)
            def _():
                base = c * _CHUNK
                pltpu.sync_copy(idxv_hbm.at[pl.ds(base, _CHUNK)], idxv_v)
                pltpu.sync_copy(idxe_hbm.at[pl.ds(base, _CHUNK)], idxe_v)
                cpn = pltpu.async_copy(nodes_hbm.at[idxv_v], nbuf, semn)
                cpe = pltpu.async_copy(edges_hbm.at[idxe_v], ebuf, seme)
                cpn.wait()
                cpe.wait()
                pltpu.sync_copy(nbuf, neigh_hbm.at[pl.ds(base, _CHUNK)])
                pltpu.sync_copy(ebuf, edg_hbm.at[pl.ds(base, _CHUNK)])

    return k


def kernel(nodes, senders, receivers, edges):
    N, DF = nodes.shape
    E, DE = edges.shape
    NSF = N * _SLOTS

    out_rank = _ranks(senders, E)
    in_rank = _ranks(receivers, E)
    out_deg = jnp.zeros((N,), jnp.int32).at[senders].add(1)
    oc = jnp.minimum(out_deg, _SLOTS)

    slot_out = out_rank
    slot_in = oc[receivers] + in_rank
    eidx = jnp.arange(E, dtype=jnp.int32)
    big = jnp.int32(NSF + 7)
    tgt_out = jnp.where(slot_out < _SLOTS, senders * _SLOTS + slot_out, big)
    tgt_in = jnp.where(slot_in < _SLOTS, receivers * _SLOTS + slot_in, big)

    flat = jnp.arange(NSF, dtype=jnp.int32)
    idxv = jnp.full((NSF,), N, jnp.int32) + (flat % 8)
    idxv = idxv.at[tgt_out].set(receivers, mode="drop")
    idxv = idxv.at[tgt_in].set(senders, mode="drop")
    idxe = jnp.full((NSF,), E, jnp.int32) + (flat % 8)
    idxe = idxe.at[tgt_out].set(eidx, mode="drop")
    idxe = idxe.at[tgt_in].set(eidx, mode="drop")

    nodes_pad = jnp.concatenate([nodes, jnp.zeros((8, DF), nodes.dtype)])
    edges_pad = jnp.concatenate([edges, jnp.zeros((8, DE), edges.dtype)])

    gk = _make_gather_kernel(N, DF, DE, NSF)
    neigh_flat, edg_flat = gk(nodes_pad, edges_pad, idxv, idxe)
    return (neigh_flat.reshape(N, _SLOTS * DF),
            edg_flat.reshape(N, _SLOTS * DE))


# speedup vs baseline: 25.5384x; 13.0235x over previous
import functools

import jax
import jax.numpy as jnp
from jax import lax
from jax.experimental import pallas as pl
from jax.experimental.pallas import tpu as pltpu
from jax.experimental.pallas import tpu_sc as plsc

_SLOTS = 4
_NW = 32
_CHUNK = 80
_L = 16
_SCAN_BASE = 1

_PARAMS = pltpu.CompilerParams(use_tc_tiling_on_sc=False,
                               needs_layout_passes=False)


def _mesh():
    return plsc.VectorSubcoreMesh(core_axis_name="c", subcore_axis_name="s")


def _wid():
    return lax.axis_index("s") * 2 + lax.axis_index("c")


def _iota():
    return lax.iota(jnp.int32, _L)


def _make_hist_kernel(E, NP):
    CH = E // _NW
    nvec = CH // _L

    @functools.partial(
        pl.kernel,
        out_type=(
            jax.ShapeDtypeStruct((_NW, NP), jnp.int32),
            jax.ShapeDtypeStruct((_NW, NP), jnp.int32),
        ),
        mesh=_mesh(),
        compiler_params=_PARAMS,
        scratch_types=[
            pltpu.VMEM((CH,), jnp.int32),
            pltpu.VMEM((CH,), jnp.int32),
            pltpu.VMEM((NP,), jnp.int32),
            pltpu.VMEM((NP,), jnp.int32),
        ],
    )
    def ka(s_hbm, r_hbm, hs_hbm, hr_hbm, sb, rb, hsv, hrv):
        w = _wid()
        pltpu.sync_copy(s_hbm.at[pl.ds(w * CH, CH)], sb)
        pltpu.sync_copy(r_hbm.at[pl.ds(w * CH, CH)], rb)

        def zbody(i, _):
            z = jnp.zeros((_L,), jnp.int32)
            hsv[pl.ds(i * _L, _L)] = z
            hrv[pl.ds(i * _L, _L)] = z
            return 0

        lax.fori_loop(0, NP // _L, zbody, 0)

        def body(i, _):
            ids = sb[pl.ds(i * _L, _L)]
            cnt, last = plsc.scan_count(ids)
            plsc.addupdate_scatter(hsv.at[...], [ids],
                                   cnt + (1 - _SCAN_BASE), mask=last)
            idr = rb[pl.ds(i * _L, _L)]
            cnt2, last2 = plsc.scan_count(idr)
            plsc.addupdate_scatter(hrv.at[...], [idr],
                                   cnt2 + (1 - _SCAN_BASE), mask=last2)
            return 0

        lax.fori_loop(0, nvec, body, 0)
        pltpu.sync_copy(hsv, hs_hbm.at[w])
        pltpu.sync_copy(hrv, hr_hbm.at[w])

    return ka


def _make_base_kernel(NP):
    NB = NP // _NW

    @functools.partial(
        pl.kernel,
        out_type=(
            jax.ShapeDtypeStruct((_NW, NP), jnp.int32),
            jax.ShapeDtypeStruct((_NW, NP), jnp.int32),
            jax.ShapeDtypeStruct((NP * _SLOTS,), jnp.int32),
        ),
        mesh=_mesh(),
        compiler_params=_PARAMS,
        scratch_types=[
            pltpu.VMEM((_NW, NB), jnp.int32),
            pltpu.VMEM((_NW, NB), jnp.int32),
            pltpu.VMEM((_NW, NB), jnp.int32),
            pltpu.VMEM((_NW, NB), jnp.int32),
            pltpu.VMEM((NB * _SLOTS,), jnp.int32),
        ],
    )
    def kb(hs_hbm, hr_hbm, bs_hbm, br_hbm, vm_hbm, hsb, hrb, bsb, brb, vmb):
        w = _wid()
        for c in range(_NW):
            pltpu.sync_copy(hs_hbm.at[c, pl.ds(w * NB, NB)], hsb.at[c])
            pltpu.sync_copy(hr_hbm.at[c, pl.ds(w * NB, NB)], hrb.at[c])
        gdims = lax.GatherDimensionNumbers(
            offset_dims=(), collapsed_slice_dims=(0,), start_index_map=(0,))
        for j in range(NB // _L):
            sl = pl.ds(j * _L, _L)
            acc = jnp.zeros((_L,), jnp.int32)
            for c in range(_NW):
                bsb[c, sl] = acc
                acc = acc + hsb[c, sl]
            oc = jnp.minimum(acc, _SLOTS)
            acc2 = oc
            for c in range(_NW):
                brb[c, sl] = acc2
                acc2 = acc2 + hrb[c, sl]
            fill = jnp.minimum(acc2, _SLOTS)
            lane = _iota()
            for q in range(_SLOTS):
                nidx = (lane // _SLOTS + _SLOTS * q).reshape(_L, 1)
                sub = lax.gather(fill, nidx, gdims, (1,),
                                 mode=lax.GatherScatterMode.PROMISE_IN_BOUNDS)
                vmb[pl.ds(j * _L * _SLOTS + q * _L, _L)] = jnp.where(
                    lane % _SLOTS < sub, 1, 0).astype(jnp.int32)
        for c in range(_NW):
            pltpu.sync_copy(bsb.at[c], bs_hbm.at[c, pl.ds(w * NB, NB)])
            pltpu.sync_copy(brb.at[c], br_hbm.at[c, pl.ds(w * NB, NB)])
        pltpu.sync_copy(vmb, vm_hbm.at[pl.ds(w * NB * _SLOTS, NB * _SLOTS)])

    return kb


def _make_select_kernel(E, NP, NSF):
    CH = E // _NW
    nvec = CH // _L
    BL = 256
    CAP = 2 * CH + 2 * BL
    NBLK = CAP // BL

    @functools.partial(
        pl.kernel,
        out_type=(
            jax.ShapeDtypeStruct((NSF + 64,), jnp.int32),
            jax.ShapeDtypeStruct((NSF + 64,), jnp.int32),
        ),
        mesh=_mesh(),
        compiler_params=_PARAMS,
        scratch_types=[
            pltpu.VMEM((CH,), jnp.int32),
            pltpu.VMEM((CH,), jnp.int32),
            pltpu.VMEM((NP,), jnp.int32),
            pltpu.VMEM((NP,), jnp.int32),
            pltpu.VMEM((CAP,), jnp.int32),
            pltpu.VMEM((CAP,), jnp.int32),
            pltpu.VMEM((CAP,), jnp.int32),
            pltpu.SemaphoreType.DMA,
        ],
    )
    def kc(s_hbm, r_hbm, bs_hbm, br_hbm, idxv_hbm, idxe_hbm,
           sb, rb, cs, cr, posb, vvb, veb, sem):
        w = _wid()
        pltpu.sync_copy(s_hbm.at[pl.ds(w * CH, CH)], sb)
        pltpu.sync_copy(r_hbm.at[pl.ds(w * CH, CH)], rb)
        pltpu.sync_copy(bs_hbm.at[w], cs)
        pltpu.sync_copy(br_hbm.at[w], cr)
        lane = _iota()

        def body(i, off):
            ids = sb[pl.ds(i * _L, _L)]
            idr = rb[pl.ds(i * _L, _L)]
            eid = w * CH + i * _L + lane

            pre = plsc.load_gather(cs.at[...], [ids])
            cnt, last = plsc.scan_count(ids)
            rank = pre + cnt - _SCAN_BASE
            plsc.store_scatter(cs.at[...], [ids],
                               pre + cnt + (1 - _SCAN_BASE), mask=last)
            valid = rank < _SLOTS
            pos = ids * _SLOTS + rank
            plsc.store_compressed(posb.at[pl.ds(off, _L)], pos, mask=valid)
            plsc.store_compressed(vvb.at[pl.ds(off, _L)], idr, mask=valid)
            plsc.store_compressed(veb.at[pl.ds(off, _L)], eid, mask=valid)
            off = off + plsc.all_reduce_population_count(valid)[0]

            pre2 = plsc.load_gather(cr.at[...], [idr])
            cnt2, last2 = plsc.scan_count(idr)
            slot = pre2 + cnt2 - _SCAN_BASE
            plsc.store_scatter(cr.at[...], [idr],
                               pre2 + cnt2 + (1 - _SCAN_BASE), mask=last2)
            valid2 = slot < _SLOTS
            pos2 = idr * _SLOTS + slot
            plsc.store_compressed(posb.at[pl.ds(off, _L)], pos2, mask=valid2)
            plsc.store_compressed(vvb.at[pl.ds(off, _L)], ids, mask=valid2)
            plsc.store_compressed(veb.at[pl.ds(off, _L)], eid, mask=valid2)
            off = off + plsc.all_reduce_population_count(valid2)[0]
            return off

        off = lax.fori_loop(0, nvec, body, jnp.int32(0))

        dump = NSF + (lane % 8) * 8
        for k in range(BL // _L):
            plsc.store_scatter(posb.at[...], [off + k * _L + lane], dump)

        nblk = (off + BL - 1) // BL

        def scat(t, _):
            sl = pl.ds(t * BL, BL)
            cpv = pltpu.async_copy(vvb.at[sl], idxv_hbm.at[posb.at[sl]], sem)
            cpe = pltpu.async_copy(veb.at[sl], idxe_hbm.at[posb.at[sl]], sem)
            cpv.wait()
            cpe.wait()
            return 0

        lax.fori_loop(0, nblk, scat, 0)

    return kc


def _make_gather_kernel(N, DF, DE, NSF):
    nchunks = NSF // _CHUNK
    iters = (nchunks + _NW - 1) // _NW

    @functools.partial(
        pl.kernel,
        out_type=(
            jax.ShapeDtypeStruct((NSF, DF), jnp.float32),
            jax.ShapeDtypeStruct((NSF, DE), jnp.float32),
        ),
        mesh=_mesh(),
        compiler_params=_PARAMS,
        scratch_types=[
            pltpu.VMEM((_CHUNK,), jnp.int32),
            pltpu.VMEM((_CHUNK,), jnp.int32),
            pltpu.VMEM((_CHUNK,), jnp.int32),
            pltpu.VMEM((_CHUNK, DF), jnp.float32),
            pltpu.VMEM((_CHUNK, DE), jnp.float32),
            pltpu.SemaphoreType.DMA,
            pltpu.SemaphoreType.DMA,
        ],
    )
    def kd(nodes_hbm, edges_hbm, idxv_hbm, idxe_hbm, vm_hbm,
           neigh_hbm, edg_hbm, idxv_v, idxe_v, vm_v, nbuf, ebuf, semn, seme):
        E8 = edges_hbm.shape[0] - 8
        wid = _wid()
        lane = _iota()
        for it in range(iters):
            c = wid + _NW * it

            @pl.when(c < nchunks)
            def _():
                base = c * _CHUNK
                pltpu.sync_copy(idxv_hbm.at[pl.ds(base, _CHUNK)], idxv_v)
                pltpu.sync_copy(idxe_hbm.at[pl.ds(base, _CHUNK)], idxe_v)
                pltpu.sync_copy(vm_hbm.at[pl.ds(base, _CHUNK)], vm_v)
                for j in range(_CHUNK // _L):
                    sl = pl.ds(j * _L, _L)
                    ok = vm_v[sl] != 0
                    idxv_v[sl] = jnp.where(ok, idxv_v[sl], N + lane % 8)
                    idxe_v[sl] = jnp.where(ok, idxe_v[sl], E8 + lane % 8)
                cpn = pltpu.async_copy(nodes_hbm.at[idxv_v], nbuf, semn)
                cpe = pltpu.async_copy(edges_hbm.at[idxe_v], ebuf, seme)
                cpn.wait()
                cpe.wait()
                pltpu.sync_copy(nbuf, neigh_hbm.at[pl.ds(base, _CHUNK)])
                pltpu.sync_copy(ebuf, edg_hbm.at[pl.ds(base, _CHUNK)])

    return kd


def kernel(nodes, senders, receivers, edges):
    N, DF = nodes.shape
    E, DE = edges.shape
    NSF = N * _SLOTS
    NP = ((N + _NW * _L - 1) // (_NW * _L)) * (_NW * _L)

    ka = _make_hist_kernel(E, NP)
    kb = _make_base_kernel(NP)
    kc = _make_select_kernel(E, NP, NSF)
    kd = _make_gather_kernel(N, DF, DE, NSF)

    hs, hr = ka(senders, receivers)
    bs, br, vm = kb(hs, hr)
    idxv, idxe = kc(senders, receivers, bs, br)

    nodes_pad = jnp.concatenate([nodes, jnp.zeros((8, DF), nodes.dtype)])
    edges_pad = jnp.concatenate([edges, jnp.zeros((8, DE), edges.dtype)])
    neigh_flat, edg_flat = kd(nodes_pad, edges_pad, idxv, idxe, vm)
    return (neigh_flat.reshape(N, _SLOTS * DF),
            edg_flat.reshape(N, _SLOTS * DE))


# speedup vs baseline: 47.6870x; 1.8673x over previous
import functools

import jax
import jax.numpy as jnp
from jax import lax
from jax.experimental import pallas as pl
from jax.experimental.pallas import tpu as pltpu
from jax.experimental.pallas import tpu_sc as plsc

_SLOTS = 4
_NW = 32
_CHUNK = 80
_L = 16
_SCAN_BASE = 1

_PARAMS = pltpu.CompilerParams(use_tc_tiling_on_sc=False,
                               needs_layout_passes=False)


def _mesh():
    return plsc.VectorSubcoreMesh(core_axis_name="c", subcore_axis_name="s")


def _wid():
    return lax.axis_index("s") * 2 + lax.axis_index("c")


def _iota():
    return lax.iota(jnp.int32, _L)


def _make_hist_kernel(E, NP):
    CH = E // _NW
    nvec = CH // _L

    @functools.partial(
        pl.kernel,
        out_type=(
            jax.ShapeDtypeStruct((_NW, NP), jnp.int32),
            jax.ShapeDtypeStruct((_NW, NP), jnp.int32),
        ),
        mesh=_mesh(),
        compiler_params=_PARAMS,
        scratch_types=[
            pltpu.VMEM((CH,), jnp.int32),
            pltpu.VMEM((CH,), jnp.int32),
            pltpu.VMEM((NP,), jnp.int32),
            pltpu.VMEM((NP,), jnp.int32),
        ],
    )
    def ka(s_hbm, r_hbm, hs_hbm, hr_hbm, sb, rb, hsv, hrv):
        w = _wid()
        pltpu.sync_copy(s_hbm.at[pl.ds(w * CH, CH)], sb)
        pltpu.sync_copy(r_hbm.at[pl.ds(w * CH, CH)], rb)

        def zbody(i, _):
            z = jnp.zeros((_L,), jnp.int32)
            hsv[pl.ds(i * _L, _L)] = z
            hrv[pl.ds(i * _L, _L)] = z
            return 0

        lax.fori_loop(0, NP // _L, zbody, 0)

        def body(i, _):
            ids = sb[pl.ds(i * _L, _L)]
            cnt, last = plsc.scan_count(ids)
            plsc.addupdate_scatter(hsv.at[...], [ids],
                                   cnt + (1 - _SCAN_BASE), mask=last)
            idr = rb[pl.ds(i * _L, _L)]
            cnt2, last2 = plsc.scan_count(idr)
            plsc.addupdate_scatter(hrv.at[...], [idr],
                                   cnt2 + (1 - _SCAN_BASE), mask=last2)
            return 0

        lax.fori_loop(0, nvec, body, 0)
        pltpu.sync_copy(hsv, hs_hbm.at[w])
        pltpu.sync_copy(hrv, hr_hbm.at[w])

    return ka


def _make_base_kernel(NP):
    NB = NP // _NW

    @functools.partial(
        pl.kernel,
        out_type=(
            jax.ShapeDtypeStruct((_NW, NP), jnp.int32),
            jax.ShapeDtypeStruct((_NW, NP), jnp.int32),
        ),
        mesh=_mesh(),
        compiler_params=_PARAMS,
        scratch_types=[
            pltpu.VMEM((_NW, NB), jnp.int32),
            pltpu.VMEM((_NW, NB), jnp.int32),
            pltpu.VMEM((_NW, NB), jnp.int32),
            pltpu.VMEM((_NW, NB), jnp.int32),
        ],
    )
    def kb(hs_hbm, hr_hbm, bs_hbm, br_hbm, hsb, hrb, bsb, brb):
        w = _wid()
        for c in range(_NW):
            pltpu.sync_copy(hs_hbm.at[c, pl.ds(w * NB, NB)], hsb.at[c])
            pltpu.sync_copy(hr_hbm.at[c, pl.ds(w * NB, NB)], hrb.at[c])
        for j in range(NB // _L):
            sl = pl.ds(j * _L, _L)
            acc = jnp.zeros((_L,), jnp.int32)
            for c in range(_NW):
                bsb[c, sl] = acc
                acc = acc + hsb[c, sl]
            oc = jnp.minimum(acc, _SLOTS)
            acc2 = oc
            for c in range(_NW):
                brb[c, sl] = acc2
                acc2 = acc2 + hrb[c, sl]
        for c in range(_NW):
            pltpu.sync_copy(bsb.at[c], bs_hbm.at[c, pl.ds(w * NB, NB)])
            pltpu.sync_copy(brb.at[c], br_hbm.at[c, pl.ds(w * NB, NB)])

    return kb


def _make_select_kernel(E, NP, NSF):
    CH = E // _NW
    nvec = CH // _L

    @functools.partial(
        pl.kernel,
        out_type=(
            jax.ShapeDtypeStruct((_NW, NSF), jnp.int32),
            jax.ShapeDtypeStruct((_NW, NSF), jnp.int32),
        ),
        mesh=_mesh(),
        compiler_params=_PARAMS,
        scratch_types=[
            pltpu.VMEM((CH,), jnp.int32),
            pltpu.VMEM((CH,), jnp.int32),
            pltpu.VMEM((NP,), jnp.int32),
            pltpu.VMEM((NP,), jnp.int32),
            pltpu.VMEM((NSF,), jnp.int32),
            pltpu.VMEM((NSF,), jnp.int32),
        ],
    )
    def kc(s_hbm, r_hbm, bs_hbm, br_hbm, idxv_hbm, idxe_hbm,
           sb, rb, cs, cr, lv, le):
        w = _wid()
        pltpu.sync_copy(s_hbm.at[pl.ds(w * CH, CH)], sb)
        pltpu.sync_copy(r_hbm.at[pl.ds(w * CH, CH)], rb)
        pltpu.sync_copy(bs_hbm.at[w], cs)
        pltpu.sync_copy(br_hbm.at[w], cr)
        lane = _iota()

        def zbody(i, _):
            z = jnp.zeros((_L,), jnp.int32)
            lv[pl.ds(i * _L, _L)] = z
            le[pl.ds(i * _L, _L)] = z
            return 0

        lax.fori_loop(0, NSF // _L, zbody, 0)

        def body(i, _):
            ids = sb[pl.ds(i * _L, _L)]
            idr = rb[pl.ds(i * _L, _L)]
            eid = w * CH + i * _L + lane

            pre = plsc.load_gather(cs.at[...], [ids])
            cnt, last = plsc.scan_count(ids)
            rank = pre + cnt - _SCAN_BASE
            plsc.store_scatter(cs.at[...], [ids],
                               pre + cnt + (1 - _SCAN_BASE), mask=last)
            valid = rank < _SLOTS
            pos = ids * _SLOTS + jnp.minimum(rank, _SLOTS - 1)
            plsc.store_scatter(lv.at[...], [pos], idr + 1, mask=valid)
            plsc.store_scatter(le.at[...], [pos], eid + 1, mask=valid)

            pre2 = plsc.load_gather(cr.at[...], [idr])
            cnt2, last2 = plsc.scan_count(idr)
            slot = pre2 + cnt2 - _SCAN_BASE
            plsc.store_scatter(cr.at[...], [idr],
                               pre2 + cnt2 + (1 - _SCAN_BASE), mask=last2)
            valid2 = slot < _SLOTS
            pos2 = idr * _SLOTS + jnp.minimum(slot, _SLOTS - 1)
            plsc.store_scatter(lv.at[...], [pos2], ids + 1, mask=valid2)
            plsc.store_scatter(le.at[...], [pos2], eid + 1, mask=valid2)
            return 0

        lax.fori_loop(0, nvec, body, 0)
        pltpu.sync_copy(lv, idxv_hbm.at[w])
        pltpu.sync_copy(le, idxe_hbm.at[w])

    return kc


def _make_gather_kernel(N, DF, DE, NSF):
    nchunks = NSF // _CHUNK
    iters = (nchunks + _NW - 1) // _NW

    @functools.partial(
        pl.kernel,
        out_type=(
            jax.ShapeDtypeStruct((NSF, DF), jnp.float32),
            jax.ShapeDtypeStruct((NSF, DE), jnp.float32),
        ),
        mesh=_mesh(),
        compiler_params=_PARAMS,
        scratch_types=[
            pltpu.VMEM((_NW, _CHUNK), jnp.int32),
            pltpu.VMEM((_NW, _CHUNK), jnp.int32),
            pltpu.VMEM((_CHUNK,), jnp.int32),
            pltpu.VMEM((_CHUNK,), jnp.int32),
            pltpu.VMEM((_CHUNK, DF), jnp.float32),
            pltpu.VMEM((_CHUNK, DE), jnp.float32),
            pltpu.SemaphoreType.DMA,
            pltpu.SemaphoreType.DMA,
        ],
    )
    def kd(nodes_hbm, edges_hbm, idxv_hbm, idxe_hbm,
           neigh_hbm, edg_hbm, pv, pe, idxv_v, idxe_v, nbuf, ebuf,
           semn, seme):
        E8 = edges_hbm.shape[0] - 8
        wid = _wid()
        lane = _iota()
        for it in range(iters):
            c = wid + _NW * it

            @pl.when(c < nchunks)
            def _():
                base = c * _CHUNK
                pltpu.sync_copy(idxv_hbm.at[:, pl.ds(base, _CHUNK)], pv)
                pltpu.sync_copy(idxe_hbm.at[:, pl.ds(base, _CHUNK)], pe)
                for j in range(_CHUNK // _L):
                    sl = pl.ds(j * _L, _L)
                    av = jnp.zeros((_L,), jnp.int32)
                    ae = jnp.zeros((_L,), jnp.int32)
                    for w in range(_NW):
                        av = av + pv[w, sl]
                        ae = ae + pe[w, sl]
                    ok = av > 0
                    idxv_v[sl] = jnp.where(ok, av - 1, N + lane % 8)
                    idxe_v[sl] = jnp.where(ok, ae - 1, E8 + lane % 8)
                cpn = pltpu.async_copy(nodes_hbm.at[idxv_v], nbuf, semn)
                cpe = pltpu.async_copy(edges_hbm.at[idxe_v], ebuf, seme)
                cpn.wait()
                cpe.wait()
                pltpu.sync_copy(nbuf, neigh_hbm.at[pl.ds(base, _CHUNK)])
                pltpu.sync_copy(ebuf, edg_hbm.at[pl.ds(base, _CHUNK)])

    return kd


def kernel(nodes, senders, receivers, edges):
    N, DF = nodes.shape
    E, DE = edges.shape
    NSF = N * _SLOTS
    NP = ((N + _NW * _L - 1) // (_NW * _L)) * (_NW * _L)

    ka = _make_hist_kernel(E, NP)
    kb = _make_base_kernel(NP)
    kc = _make_select_kernel(E, NP, NSF)
    kd = _make_gather_kernel(N, DF, DE, NSF)

    hs, hr = ka(senders, receivers)
    bs, br = kb(hs, hr)
    idxv, idxe = kc(senders, receivers, bs, br)

    nodes_pad = jnp.concatenate([nodes, jnp.zeros((8, DF), nodes.dtype)])
    edges_pad = jnp.concatenate([edges, jnp.zeros((8, DE), edges.dtype)])
    neigh_flat, edg_flat = kd(nodes_pad, edges_pad, idxv, idxe)
    return (neigh_flat.reshape(N, _SLOTS * DF),
            edg_flat.reshape(N, _SLOTS * DE))


# speedup vs baseline: 55.8818x; 1.1718x over previous
import functools

import jax
import jax.numpy as jnp
from jax import lax
from jax.experimental import pallas as pl
from jax.experimental.pallas import tpu as pltpu
from jax.experimental.pallas import tpu_sc as plsc

_SLOTS = 4
_NW = 32
_CHUNK = 80
_L = 16
_SCAN_BASE = 1

_PARAMS = pltpu.CompilerParams(use_tc_tiling_on_sc=False,
                               needs_layout_passes=False)


def _mesh():
    return plsc.VectorSubcoreMesh(core_axis_name="c", subcore_axis_name="s")


def _wid():
    return lax.axis_index("s") * 2 + lax.axis_index("c")


def _iota():
    return lax.iota(jnp.int32, _L)


def _make_hist_kernel(E, NP):
    CH = E // _NW
    nvec = CH // _L

    @functools.partial(
        pl.kernel,
        out_type=(
            jax.ShapeDtypeStruct((_NW, NP), jnp.int32),
            jax.ShapeDtypeStruct((_NW, NP), jnp.int32),
        ),
        mesh=_mesh(),
        compiler_params=_PARAMS,
        scratch_types=[
            pltpu.VMEM((CH,), jnp.int32),
            pltpu.VMEM((CH,), jnp.int32),
            pltpu.VMEM((NP,), jnp.int32),
            pltpu.VMEM((NP,), jnp.int32),
        ],
    )
    def ka(s_hbm, r_hbm, hs_hbm, hr_hbm, sb, rb, hsv, hrv):
        w = _wid()
        pltpu.sync_copy(s_hbm.at[pl.ds(w * CH, CH)], sb)
        pltpu.sync_copy(r_hbm.at[pl.ds(w * CH, CH)], rb)

        def zbody(i, _):
            z = jnp.zeros((_L,), jnp.int32)
            hsv[pl.ds(i * _L, _L)] = z
            hrv[pl.ds(i * _L, _L)] = z
            return 0

        lax.fori_loop(0, NP // _L, zbody, 0)

        def body(i, _):
            ids = sb[pl.ds(i * _L, _L)]
            cnt, last = plsc.scan_count(ids)
            plsc.addupdate_scatter(hsv.at[...], [ids],
                                   cnt + (1 - _SCAN_BASE), mask=last)
            idr = rb[pl.ds(i * _L, _L)]
            cnt2, last2 = plsc.scan_count(idr)
            plsc.addupdate_scatter(hrv.at[...], [idr],
                                   cnt2 + (1 - _SCAN_BASE), mask=last2)
            return 0

        lax.fori_loop(0, nvec, body, 0)
        pltpu.sync_copy(hsv, hs_hbm.at[w])
        pltpu.sync_copy(hrv, hr_hbm.at[w])

    return ka


def _make_base_kernel(NP):
    NB = NP // _NW

    @functools.partial(
        pl.kernel,
        out_type=(
            jax.ShapeDtypeStruct((_NW, NP), jnp.int32),
            jax.ShapeDtypeStruct((_NW, NP), jnp.int32),
        ),
        mesh=_mesh(),
        compiler_params=_PARAMS,
        scratch_types=[
            pltpu.VMEM((_NW, NB), jnp.int32),
            pltpu.VMEM((_NW, NB), jnp.int32),
            pltpu.VMEM((_NW, NB), jnp.int32),
            pltpu.VMEM((_NW, NB), jnp.int32),
        ],
    )
    def kb(hs_hbm, hr_hbm, bs_hbm, br_hbm, hsb, hrb, bsb, brb):
        w = _wid()
        pltpu.sync_copy(hs_hbm.at[:, pl.ds(w * NB, NB)], hsb)
        pltpu.sync_copy(hr_hbm.at[:, pl.ds(w * NB, NB)], hrb)
        for j in range(NB // _L):
            sl = pl.ds(j * _L, _L)
            acc = jnp.zeros((_L,), jnp.int32)
            for c in range(_NW):
                bsb[c, sl] = acc
                acc = acc + hsb[c, sl]
            oc = jnp.minimum(acc, _SLOTS)
            acc2 = oc
            for c in range(_NW):
                brb[c, sl] = acc2
                acc2 = acc2 + hrb[c, sl]
        pltpu.sync_copy(bsb, bs_hbm.at[:, pl.ds(w * NB, NB)])
        pltpu.sync_copy(brb, br_hbm.at[:, pl.ds(w * NB, NB)])

    return kb


def _make_select_kernel(E, NP, NSF):
    CH = E // _NW
    nvec = CH // _L

    @functools.partial(
        pl.kernel,
        out_type=(
            jax.ShapeDtypeStruct((_NW, NSF), jnp.int32),
            jax.ShapeDtypeStruct((_NW, NSF), jnp.int32),
        ),
        mesh=_mesh(),
        compiler_params=_PARAMS,
        scratch_types=[
            pltpu.VMEM((CH,), jnp.int32),
            pltpu.VMEM((CH,), jnp.int32),
            pltpu.VMEM((NP,), jnp.int32),
            pltpu.VMEM((NP,), jnp.int32),
            pltpu.VMEM((NSF,), jnp.int32),
            pltpu.VMEM((NSF,), jnp.int32),
        ],
    )
    def kc(s_hbm, r_hbm, bs_hbm, br_hbm, idxv_hbm, idxe_hbm,
           sb, rb, cs, cr, lv, le):
        w = _wid()
        pltpu.sync_copy(s_hbm.at[pl.ds(w * CH, CH)], sb)
        pltpu.sync_copy(r_hbm.at[pl.ds(w * CH, CH)], rb)
        pltpu.sync_copy(bs_hbm.at[w], cs)
        pltpu.sync_copy(br_hbm.at[w], cr)
        lane = _iota()

        def zbody(i, _):
            z = jnp.zeros((_L,), jnp.int32)
            for u in range(5):
                lv[pl.ds(i * 5 * _L + u * _L, _L)] = z
                le[pl.ds(i * 5 * _L + u * _L, _L)] = z
            return 0

        lax.fori_loop(0, NSF // (5 * _L), zbody, 0)

        def body(i, _):
            ids = sb[pl.ds(i * _L, _L)]
            idr = rb[pl.ds(i * _L, _L)]
            eid = w * CH + i * _L + lane

            pre = plsc.load_gather(cs.at[...], [ids])
            cnt, last = plsc.scan_count(ids)
            rank = pre + cnt - _SCAN_BASE
            plsc.store_scatter(cs.at[...], [ids],
                               pre + cnt + (1 - _SCAN_BASE), mask=last)
            valid = rank < _SLOTS
            pos = ids * _SLOTS + jnp.minimum(rank, _SLOTS - 1)
            plsc.store_scatter(lv.at[...], [pos], idr + 1, mask=valid)
            plsc.store_scatter(le.at[...], [pos], eid + 1, mask=valid)

            pre2 = plsc.load_gather(cr.at[...], [idr])
            cnt2, last2 = plsc.scan_count(idr)
            slot = pre2 + cnt2 - _SCAN_BASE
            plsc.store_scatter(cr.at[...], [idr],
                               pre2 + cnt2 + (1 - _SCAN_BASE), mask=last2)
            valid2 = slot < _SLOTS
            pos2 = idr * _SLOTS + jnp.minimum(slot, _SLOTS - 1)
            plsc.store_scatter(lv.at[...], [pos2], ids + 1, mask=valid2)
            plsc.store_scatter(le.at[...], [pos2], eid + 1, mask=valid2)
            return 0

        lax.fori_loop(0, nvec, body, 0)
        pltpu.sync_copy(lv, idxv_hbm.at[w])
        pltpu.sync_copy(le, idxe_hbm.at[w])

    return kc


def _make_gather_kernel(N, DF, DE, NSF):
    nchunks = NSF // _CHUNK
    iters = (nchunks + _NW - 1) // _NW

    @functools.partial(
        pl.kernel,
        out_type=(
            jax.ShapeDtypeStruct((NSF, DF), jnp.float32),
            jax.ShapeDtypeStruct((NSF, DE), jnp.float32),
        ),
        mesh=_mesh(),
        compiler_params=_PARAMS,
        scratch_types=[
            pltpu.VMEM((2, _NW, _CHUNK), jnp.int32),
            pltpu.VMEM((2, _NW, _CHUNK), jnp.int32),
            pltpu.VMEM((2, _CHUNK), jnp.int32),
            pltpu.VMEM((2, _CHUNK), jnp.int32),
            pltpu.VMEM((2, _CHUNK, DF), jnp.float32),
            pltpu.VMEM((2, _CHUNK, DE), jnp.float32),
            pltpu.SemaphoreType.DMA,
            pltpu.SemaphoreType.DMA,
            pltpu.SemaphoreType.DMA,
            pltpu.SemaphoreType.DMA,
            pltpu.SemaphoreType.DMA,
        ],
    )
    def kd(nodes_hbm, edges_hbm, idxv_hbm, idxe_hbm,
           neigh_hbm, edg_hbm, pv, pe, idxv_v, idxe_v, nbuf, ebuf,
           semi, semn, seme, semo0, semo1):
        E8 = edges_hbm.shape[0] - 8
        wid = _wid()
        lane = _iota()
        semo = [semo0, semo1]

        def chunk_base(it):
            return jnp.minimum(wid + _NW * it, nchunks - 1) * _CHUNK

        def issue_idx(it, b):
            base = chunk_base(it)
            c1 = pltpu.async_copy(idxv_hbm.at[:, pl.ds(base, _CHUNK)],
                                  pv.at[b], semi)
            c2 = pltpu.async_copy(idxe_hbm.at[:, pl.ds(base, _CHUNK)],
                                  pe.at[b], semi)
            return (c1, c2)

        pend_idx = [None, None]
        pend_out = [None, None]
        pend_idx[0] = issue_idx(0, 0)
        for it in range(iters):
            b = it % 2
            if pend_out[b] is not None:
                for cp in pend_out[b]:
                    cp.wait()
                pend_out[b] = None
            for cp in pend_idx[b]:
                cp.wait()
            if it + 1 < iters:
                pend_idx[b ^ 1] = issue_idx(it + 1, b ^ 1)
            for j in range(_CHUNK // _L):
                sl = pl.ds(j * _L, _L)
                av = jnp.zeros((_L,), jnp.int32)
                ae = jnp.zeros((_L,), jnp.int32)
                for w in range(_NW):
                    av = av + pv[b, w, sl]
                    ae = ae + pe[b, w, sl]
                ok = av > 0
                idxv_v[b, sl] = jnp.where(ok, av - 1, N + lane % 8)
                idxe_v[b, sl] = jnp.where(ok, ae - 1, E8 + lane % 8)
            cpn = pltpu.async_copy(nodes_hbm.at[idxv_v.at[b]], nbuf.at[b],
                                   semn)
            cpe = pltpu.async_copy(edges_hbm.at[idxe_v.at[b]], ebuf.at[b],
                                   seme)
            cpn.wait()
            cpe.wait()
            base = chunk_base(it)
            o1 = pltpu.async_copy(nbuf.at[b],
                                  neigh_hbm.at[pl.ds(base, _CHUNK)], semo[b])
            o2 = pltpu.async_copy(ebuf.at[b],
                                  edg_hbm.at[pl.ds(base, _CHUNK)], semo[b])
            pend_out[b] = (o1, o2)
        for b in range(2):
            if pend_out[b] is not None:
                for cp in pend_out[b]:
                    cp.wait()

    return kd


def kernel(nodes, senders, receivers, edges):
    N, DF = nodes.shape
    E, DE = edges.shape
    NSF = N * _SLOTS
    NP = ((N + _NW * _L - 1) // (_NW * _L)) * (_NW * _L)

    ka = _make_hist_kernel(E, NP)
    kb = _make_base_kernel(NP)
    kc = _make_select_kernel(E, NP, NSF)
    kd = _make_gather_kernel(N, DF, DE, NSF)

    hs, hr = ka(senders, receivers)
    bs, br = kb(hs, hr)
    idxv, idxe = kc(senders, receivers, bs, br)

    nodes_pad = jnp.concatenate([nodes, jnp.zeros((8, DF), nodes.dtype)])
    edges_pad = jnp.concatenate([edges, jnp.zeros((8, DE), edges.dtype)])
    neigh_flat, edg_flat = kd(nodes_pad, edges_pad, idxv, idxe)
    return (neigh_flat.reshape(N, _SLOTS * DF),
            edg_flat.reshape(N, _SLOTS * DE))


# speedup vs baseline: 68.8147x; 1.2314x over previous
import functools

import jax
import jax.numpy as jnp
from jax import lax
from jax.experimental import pallas as pl
from jax.experimental.pallas import tpu as pltpu
from jax.experimental.pallas import tpu_sc as plsc

_SLOTS = 4
_NW = 32
_CHUNK = 80
_L = 16
_SCAN_BASE = 1

_PARAMS = pltpu.CompilerParams(use_tc_tiling_on_sc=False,
                               needs_layout_passes=False)


def _mesh():
    return plsc.VectorSubcoreMesh(core_axis_name="c", subcore_axis_name="s")


def _wid():
    return lax.axis_index("s") * 2 + lax.axis_index("c")


def _iota():
    return lax.iota(jnp.int32, _L)


def _make_hist_kernel(E, NP, N, DF, DE):
    CH = E // _NW
    nvec = CH // _L
    WN = N * DF
    WE = E * DE
    T = 20000
    NT_N = WN // _NW // T
    NT_E = WE // _NW // T
    NT = NT_N + NT_E

    @functools.partial(
        pl.kernel,
        out_type=(
            jax.ShapeDtypeStruct((_NW, NP), jnp.int32),
            jax.ShapeDtypeStruct((_NW, NP), jnp.int32),
            jax.ShapeDtypeStruct((WN + 8 * DF,), jnp.float32),
            jax.ShapeDtypeStruct((WE + 8 * DE,), jnp.float32),
        ),
        mesh=_mesh(),
        compiler_params=_PARAMS,
        scratch_types=[
            pltpu.VMEM((CH,), jnp.int32),
            pltpu.VMEM((CH,), jnp.int32),
            pltpu.VMEM((NP,), jnp.int32),
            pltpu.VMEM((NP,), jnp.int32),
            pltpu.VMEM((2, T), jnp.float32),
            pltpu.VMEM((8 * DF,), jnp.float32),
            pltpu.SemaphoreType.DMA,
            pltpu.SemaphoreType.DMA,
            pltpu.SemaphoreType.DMA,
            pltpu.SemaphoreType.DMA,
        ],
    )
    def ka(s_hbm, r_hbm, nf_hbm, ef_hbm, hs_hbm, hr_hbm, np_hbm, ep_hbm,
           sb, rb, hsv, hrv, tb, zb, semsr, semi, semo0, semo1):
        w = _wid()
        cp_s = pltpu.async_copy(s_hbm.at[pl.ds(w * CH, CH)], sb, semsr)
        cp_r = pltpu.async_copy(r_hbm.at[pl.ds(w * CH, CH)], rb, semsr)

        def tile_refs(k):
            if k < NT_N:
                off = w * (NT_N * T) + k * T
                return nf_hbm.at[pl.ds(off, T)], np_hbm.at[pl.ds(off, T)]
            off = w * (NT_E * T) + (k - NT_N) * T
            return ef_hbm.at[pl.ds(off, T)], ep_hbm.at[pl.ds(off, T)]

        semo = [semo0, semo1]
        pend_in = [None, None]
        pend_out = [None, None]
        pend_in[0] = pltpu.async_copy(tile_refs(0)[0], tb.at[0], semi)

        cp_s.wait()
        cp_r.wait()

        def zbody(i, _):
            z = jnp.zeros((_L,), jnp.int32)
            hsv[pl.ds(i * _L, _L)] = z
            hrv[pl.ds(i * _L, _L)] = z
            return 0

        lax.fori_loop(0, NP // _L, zbody, 0)

        def body(i, _):
            ids = sb[pl.ds(i * _L, _L)]
            cnt, last = plsc.scan_count(ids)
            plsc.addupdate_scatter(hsv.at[...], [ids],
                                   cnt + (1 - _SCAN_BASE), mask=last)
            idr = rb[pl.ds(i * _L, _L)]
            cnt2, last2 = plsc.scan_count(idr)
            plsc.addupdate_scatter(hrv.at[...], [idr],
                                   cnt2 + (1 - _SCAN_BASE), mask=last2)
            return 0

        step = nvec // NT
        bounds = [i * step for i in range(NT)] + [nvec]
        for k in range(NT):
            b = k % 2
            pend_in[b].wait()
            pend_out[b] = pltpu.async_copy(tb.at[b], tile_refs(k)[1],
                                           semo[b])
            if k + 1 < NT:
                if pend_out[b ^ 1] is not None:
                    pend_out[b ^ 1].wait()
                    pend_out[b ^ 1] = None
                pend_in[b ^ 1] = pltpu.async_copy(tile_refs(k + 1)[0],
                                                  tb.at[b ^ 1], semi)
            lax.fori_loop(bounds[k], bounds[k + 1], body, 0)
        for b in range(2):
            if pend_out[b] is not None:
                pend_out[b].wait()

        @pl.when(w == 0)
        def _():
            for i in range((8 * DF) // _L):
                zb[pl.ds(i * _L, _L)] = jnp.zeros((_L,), jnp.float32)
            pltpu.sync_copy(zb, np_hbm.at[pl.ds(WN, 8 * DF)])
            pltpu.sync_copy(zb.at[pl.ds(0, 8 * DE)],
                            ep_hbm.at[pl.ds(WE, 8 * DE)])

        pltpu.sync_copy(hsv, hs_hbm.at[w])
        pltpu.sync_copy(hrv, hr_hbm.at[w])

    return ka


def _make_base_kernel(NP):
    NB = NP // _NW

    @functools.partial(
        pl.kernel,
        out_type=(
            jax.ShapeDtypeStruct((_NW, NP), jnp.int32),
            jax.ShapeDtypeStruct((_NW, NP), jnp.int32),
        ),
        mesh=_mesh(),
        compiler_params=_PARAMS,
        scratch_types=[
            pltpu.VMEM((_NW, NB), jnp.int32),
            pltpu.VMEM((_NW, NB), jnp.int32),
            pltpu.VMEM((_NW, NB), jnp.int32),
            pltpu.VMEM((_NW, NB), jnp.int32),
        ],
    )
    def kb(hs_hbm, hr_hbm, bs_hbm, br_hbm, hsb, hrb, bsb, brb):
        w = _wid()
        pltpu.sync_copy(hs_hbm.at[:, pl.ds(w * NB, NB)], hsb)
        pltpu.sync_copy(hr_hbm.at[:, pl.ds(w * NB, NB)], hrb)
        for j in range(NB // _L):
            sl = pl.ds(j * _L, _L)
            acc = jnp.zeros((_L,), jnp.int32)
            for c in range(_NW):
                bsb[c, sl] = acc
                acc = acc + hsb[c, sl]
            oc = jnp.minimum(acc, _SLOTS)
            acc2 = oc
            for c in range(_NW):
                brb[c, sl] = acc2
                acc2 = acc2 + hrb[c, sl]
        pltpu.sync_copy(bsb, bs_hbm.at[:, pl.ds(w * NB, NB)])
        pltpu.sync_copy(brb, br_hbm.at[:, pl.ds(w * NB, NB)])

    return kb


def _make_select_kernel(E, NP, NSF):
    CH = E // _NW
    nvec = CH // _L

    @functools.partial(
        pl.kernel,
        out_type=(
            jax.ShapeDtypeStruct((_NW, NSF), jnp.int32),
            jax.ShapeDtypeStruct((_NW, NSF), jnp.int32),
        ),
        mesh=_mesh(),
        compiler_params=_PARAMS,
        scratch_types=[
            pltpu.VMEM((CH,), jnp.int32),
            pltpu.VMEM((CH,), jnp.int32),
            pltpu.VMEM((NP,), jnp.int32),
            pltpu.VMEM((NP,), jnp.int32),
            pltpu.VMEM((NSF,), jnp.int32),
            pltpu.VMEM((NSF,), jnp.int32),
        ],
    )
    def kc(s_hbm, r_hbm, bs_hbm, br_hbm, idxv_hbm, idxe_hbm,
           sb, rb, cs, cr, lv, le):
        w = _wid()
        pltpu.sync_copy(s_hbm.at[pl.ds(w * CH, CH)], sb)
        pltpu.sync_copy(r_hbm.at[pl.ds(w * CH, CH)], rb)
        pltpu.sync_copy(bs_hbm.at[w], cs)
        pltpu.sync_copy(br_hbm.at[w], cr)
        lane = _iota()

        def zbody(i, _):
            z = jnp.zeros((_L,), jnp.int32)
            for u in range(5):
                lv[pl.ds(i * 5 * _L + u * _L, _L)] = z
                le[pl.ds(i * 5 * _L + u * _L, _L)] = z
            return 0

        lax.fori_loop(0, NSF // (5 * _L), zbody, 0)

        def body(i, _):
            ids = sb[pl.ds(i * _L, _L)]
            idr = rb[pl.ds(i * _L, _L)]
            eid = w * CH + i * _L + lane

            pre = plsc.load_gather(cs.at[...], [ids])
            cnt, last = plsc.scan_count(ids)
            rank = pre + cnt - _SCAN_BASE
            plsc.store_scatter(cs.at[...], [ids],
                               pre + cnt + (1 - _SCAN_BASE), mask=last)
            valid = rank < _SLOTS
            pos = ids * _SLOTS + jnp.minimum(rank, _SLOTS - 1)
            plsc.store_scatter(lv.at[...], [pos], idr + 1, mask=valid)
            plsc.store_scatter(le.at[...], [pos], eid + 1, mask=valid)

            pre2 = plsc.load_gather(cr.at[...], [idr])
            cnt2, last2 = plsc.scan_count(idr)
            slot = pre2 + cnt2 - _SCAN_BASE
            plsc.store_scatter(cr.at[...], [idr],
                               pre2 + cnt2 + (1 - _SCAN_BASE), mask=last2)
            valid2 = slot < _SLOTS
            pos2 = idr * _SLOTS + jnp.minimum(slot, _SLOTS - 1)
            plsc.store_scatter(lv.at[...], [pos2], ids + 1, mask=valid2)
            plsc.store_scatter(le.at[...], [pos2], eid + 1, mask=valid2)
            return 0

        lax.fori_loop(0, nvec, body, 0)
        pltpu.sync_copy(lv, idxv_hbm.at[w])
        pltpu.sync_copy(le, idxe_hbm.at[w])

    return kc


def _make_gather_kernel(N, DF, DE, NSF):
    nchunks = NSF // _CHUNK
    iters = (nchunks + _NW - 1) // _NW

    @functools.partial(
        pl.kernel,
        out_type=(
            jax.ShapeDtypeStruct((NSF, DF), jnp.float32),
            jax.ShapeDtypeStruct((NSF, DE), jnp.float32),
        ),
        mesh=_mesh(),
        compiler_params=_PARAMS,
        scratch_types=[
            pltpu.VMEM((2, _NW, _CHUNK), jnp.int32),
            pltpu.VMEM((2, _NW, _CHUNK), jnp.int32),
            pltpu.VMEM((2, _CHUNK), jnp.int32),
            pltpu.VMEM((2, _CHUNK), jnp.int32),
            pltpu.VMEM((2, _CHUNK, DF), jnp.float32),
            pltpu.VMEM((2, _CHUNK, DE), jnp.float32),
            pltpu.SemaphoreType.DMA,
            pltpu.SemaphoreType.DMA,
            pltpu.SemaphoreType.DMA,
            pltpu.SemaphoreType.DMA,
            pltpu.SemaphoreType.DMA,
        ],
    )
    def kd(nodes_hbm, edges_hbm, idxv_hbm, idxe_hbm,
           neigh_hbm, edg_hbm, pv, pe, idxv_v, idxe_v, nbuf, ebuf,
           semi, semn, seme, semo0, semo1):
        E8 = edges_hbm.shape[0] - 8
        wid = _wid()
        lane = _iota()
        semo = [semo0, semo1]

        def chunk_base(it):
            return jnp.minimum(wid + _NW * it, nchunks - 1) * _CHUNK

        def issue_idx(it, b):
            base = chunk_base(it)
            c1 = pltpu.async_copy(idxv_hbm.at[:, pl.ds(base, _CHUNK)],
                                  pv.at[b], semi)
            c2 = pltpu.async_copy(idxe_hbm.at[:, pl.ds(base, _CHUNK)],
                                  pe.at[b], semi)
            return (c1, c2)

        pend_idx = [None, None]
        pend_out = [None, None]
        pend_idx[0] = issue_idx(0, 0)
        for it in range(iters):
            b = it % 2
            if pend_out[b] is not None:
                for cp in pend_out[b]:
                    cp.wait()
                pend_out[b] = None
            for cp in pend_idx[b]:
                cp.wait()
            if it + 1 < iters:
                pend_idx[b ^ 1] = issue_idx(it + 1, b ^ 1)
            for j in range(_CHUNK // _L):
                sl = pl.ds(j * _L, _L)
                av = jnp.zeros((_L,), jnp.int32)
                ae = jnp.zeros((_L,), jnp.int32)
                for w in range(_NW):
                    av = av + pv[b, w, sl]
                    ae = ae + pe[b, w, sl]
                ok = av > 0
                idxv_v[b, sl] = jnp.where(ok, av - 1, N + lane % 8)
                idxe_v[b, sl] = jnp.where(ok, ae - 1, E8 + lane % 8)
            cpn = pltpu.async_copy(nodes_hbm.at[idxv_v.at[b]], nbuf.at[b],
                                   semn)
            cpe = pltpu.async_copy(edges_hbm.at[idxe_v.at[b]], ebuf.at[b],
                                   seme)
            cpn.wait()
            cpe.wait()
            base = chunk_base(it)
            o1 = pltpu.async_copy(nbuf.at[b],
                                  neigh_hbm.at[pl.ds(base, _CHUNK)], semo[b])
            o2 = pltpu.async_copy(ebuf.at[b],
                                  edg_hbm.at[pl.ds(base, _CHUNK)], semo[b])
            pend_out[b] = (o1, o2)
        for b in range(2):
            if pend_out[b] is not None:
                for cp in pend_out[b]:
                    cp.wait()

    return kd


def kernel(nodes, senders, receivers, edges):
    N, DF = nodes.shape
    E, DE = edges.shape
    NSF = N * _SLOTS
    NP = ((N + _NW * _L - 1) // (_NW * _L)) * (_NW * _L)

    ka = _make_hist_kernel(E, NP, N, DF, DE)
    kb = _make_base_kernel(NP)
    kc = _make_select_kernel(E, NP, NSF)
    kd = _make_gather_kernel(N, DF, DE, NSF)

    hs, hr, npad_flat, epad_flat = ka(senders, receivers,
                                      nodes.reshape(-1), edges.reshape(-1))
    bs, br = kb(hs, hr)
    idxv, idxe = kc(senders, receivers, bs, br)

    nodes_pad = npad_flat.reshape(N + 8, DF)
    edges_pad = epad_flat.reshape(E + 8, DE)
    neigh_flat, edg_flat = kd(nodes_pad, edges_pad, idxv, idxe)
    return (neigh_flat.reshape(N, _SLOTS * DF),
            edg_flat.reshape(N, _SLOTS * DE))
